# Initial kernel scaffold; baseline (speedup 1.0000x reference)
import jax
import jax.numpy as jnp
from jax.experimental import pallas as pl


def kernel(x, phone_duration, sil_duration, src_lens, max_len):
    raise NotImplementedError("write your pallas kernel here")


# speedup vs baseline: 13.9693x; 13.9693x over previous
import functools

import jax
import jax.numpy as jnp
from jax import lax
from jax.experimental import pallas as pl
from jax.experimental.pallas import tpu as pltpu
from jax.experimental.pallas import tpu_sc as plsc

B, L, H, M = 16, 2048, 256, 8192
HALF = M // 2
CH = 128
NCH = HALF // CH
ZROW = B * L
NLANE = 16


def _sc_body(xp, pd, sd, sl, out, tl16,
             pd_v, sd_v, sl_v, tl_v, a_ref, g_ref, buf0, buf1, sem0, sem1):
    cid = lax.axis_index("c")
    sid = lax.axis_index("s")
    wid = sid * 2 + cid
    b = wid >> 1
    half = wid & 1
    lo = half * HALF

    pltpu.sync_copy(pd.at[b], pd_v)
    pltpu.sync_copy(sd.at[b], sd_v)
    pltpu.sync_copy(sl, sl_v)
    lane = lax.iota(jnp.int32, NLANE)
    src_len = jnp.sum(jnp.where(lane == b, sl_v[...], 0))

    def init_body(i, _):
        a_ref[pl.ds(i * NLANE, NLANE)] = jnp.zeros((NLANE,), jnp.int32)
        return 0
    lax.fori_loop(0, HALF // NLANE, init_body, 0)

    def scan_body(i, carry):
        cp, cs, c0 = carry
        lvec = i * NLANE + lax.iota(jnp.int32, NLANE)
        valid = lvec < src_len
        pr = jnp.maximum(jnp.where(valid, pd_v[pl.ds(i * NLANE, NLANE)], 0), 1)
        sr = jnp.where(valid, sd_v[pl.ds(i * NLANE, NLANE)], 0)
        P = plsc.cumsum(pr) + cp
        S = plsc.cumsum(sr) + cs
        Pe = P - pr
        Se = S - sr
        key_p = 2 * lvec
        key_s = key_p + 1
        st_p = Pe + Se
        st_s = P + Se
        lp = st_p - lo
        ls = st_s - lo
        mask_p = (lp >= 0) & (lp < HALF)
        mask_s = (ls >= 0) & (ls < HALF) & (sr > 0)
        plsc.store_scatter(a_ref, [jnp.clip(lp, 0, HALF - 1)], key_p, mask=mask_p)
        plsc.store_scatter(a_ref, [jnp.clip(ls, 0, HALF - 1)], key_s, mask=mask_s)
        c0 = jnp.maximum(c0, jnp.max(jnp.where(st_p < lo, key_p, 0)))
        c0 = jnp.maximum(c0, jnp.max(jnp.where((st_s < lo) & (sr > 0), key_s, 0)))
        return cp + jnp.sum(pr), cs + jnp.sum(sr), c0

    cp, cs, c0 = lax.fori_loop(0, L // NLANE, scan_body, (0, 0, 0))
    tgt = cp + cs

    def fill_body(i, cm):
        seg = jnp.maximum(plsc.cummax(a_ref[pl.ds(i * NLANE, NLANE)]), cm)
        tvec = lo + i * NLANE + lax.iota(jnp.int32, NLANE)
        row = jnp.where((seg & 1) == 1, 0, seg >> 1)
        gid = jnp.where(tvec < tgt, b * L + row, ZROW)
        g_ref[i // (CH // NLANE), pl.ds((i % (CH // NLANE)) * NLANE, NLANE)] = gid
        return jnp.max(seg)
    lax.fori_loop(0, HALF // NLANE, fill_body, c0)

    @pl.when(half == 1)
    def _():
        tl_v[...] = jnp.full((NLANE,), tgt, jnp.int32)
        pltpu.sync_copy(tl_v, tl16.at[b])

    obase = b * M + lo

    def gather(c, buf, sem):
        pltpu.async_copy(xp.at[g_ref.at[c]], buf, sem)

    def gwait(c, buf, sem):
        pltpu.make_async_copy(xp.at[g_ref.at[c]], buf, sem).wait()

    gather(0, buf0, sem0)
    gather(1, buf1, sem1)

    def gbody(i, _):
        c0_, c1_ = 2 * i, 2 * i + 1
        gwait(c0_, buf0, sem0)
        pltpu.sync_copy(buf0, out.at[pl.ds(obase + c0_ * CH, CH)])

        @pl.when(i < NCH // 2 - 1)
        def _():
            gather(2 * i + 2, buf0, sem0)

        gwait(c1_, buf1, sem1)
        pltpu.sync_copy(buf1, out.at[pl.ds(obase + c1_ * CH, CH)])

        @pl.when(i < NCH // 2 - 1)
        def _():
            gather(2 * i + 3, buf1, sem1)
        return 0
    lax.fori_loop(0, NCH // 2, gbody, 0)


@jax.jit
def _run(xp, pd, sd, sl):
    mesh = plsc.VectorSubcoreMesh(core_axis_name="c", subcore_axis_name="s")
    f = pl.kernel(
        _sc_body,
        out_type=(
            jax.ShapeDtypeStruct((B * M, H), jnp.float32),
            jax.ShapeDtypeStruct((16, 16), jnp.int32),
        ),
        mesh=mesh,
        scratch_types=[
            pltpu.VMEM((L,), jnp.int32),
            pltpu.VMEM((L,), jnp.int32),
            pltpu.VMEM((16,), jnp.int32),
            pltpu.VMEM((16,), jnp.int32),
            pltpu.VMEM((HALF,), jnp.int32),
            pltpu.VMEM((NCH, CH), jnp.int32),
            pltpu.VMEM((CH, H), jnp.float32),
            pltpu.VMEM((CH, H), jnp.float32),
            pltpu.SemaphoreType.DMA,
            pltpu.SemaphoreType.DMA,
        ],
        compiler_params=pltpu.CompilerParams(needs_layout_passes=False),
    )
    return f(xp, pd, sd, sl)


def kernel(x, phone_duration, sil_duration, src_lens, max_len):
    xp = jnp.concatenate(
        [x.reshape(B * L, H), jnp.zeros((8, H), x.dtype)], axis=0)
    out_flat, tl16 = _run(xp, phone_duration, sil_duration, src_lens)
    return out_flat.reshape(B, M, H), tl16[:, 0]


# speedup vs baseline: 64.6323x; 4.6267x over previous
import jax
import jax.numpy as jnp
from jax import lax
from jax.experimental import pallas as pl
from jax.experimental.pallas import tpu as pltpu
from jax.experimental.pallas import tpu_sc as plsc

B, L, H, M = 16, 2048, 256, 8192
HALF = M // 2
CH = 128
NCH = HALF // CH
NLANE = 16


def _sc_body(xf, pd, sd, sl, out, tl16,
             pd_v, sd_v, sl_v, tl_v, a_ref, g_ref, buf0, buf1, zbuf,
             sem0, sem1):
    cid = lax.axis_index("c")
    sid = lax.axis_index("s")
    b = sid
    half = (sid + cid) & 1
    lo = half * HALF

    pltpu.sync_copy(pd.at[b], pd_v)
    pltpu.sync_copy(sd.at[b], sd_v)
    pltpu.sync_copy(sl, sl_v)
    lane = lax.iota(jnp.int32, NLANE)
    src_len = jnp.sum(jnp.where(lane == b, sl_v[...], 0))

    zero16 = jnp.zeros((NLANE,), jnp.int32)
    zf16 = jnp.zeros((NLANE,), jnp.float32)

    def init_body(i, _):
        a_ref[pl.ds(i * NLANE, NLANE)] = zero16
        return 0
    lax.fori_loop(0, HALF // NLANE, init_body, 0)

    def zinit_body(i, _):
        zbuf[i // (H // NLANE), pl.ds((i % (H // NLANE)) * NLANE, NLANE)] = zf16
        return 0
    lax.fori_loop(0, CH * H // NLANE, zinit_body, 0)

    def scan_body(i, carry):
        cp, cs, c0 = carry
        lvec = i * NLANE + lane
        valid = lvec < src_len
        pr = jnp.maximum(jnp.where(valid, pd_v[pl.ds(i * NLANE, NLANE)], 0), 1)
        sr = jnp.where(valid, sd_v[pl.ds(i * NLANE, NLANE)], 0)
        P = plsc.cumsum(pr) + cp
        S = plsc.cumsum(sr) + cs
        Pe = P - pr
        Se = S - sr
        key_p = 2 * lvec
        key_s = key_p + 1
        st_p = Pe + Se
        st_s = P + Se
        lp = st_p - lo
        ls = st_s - lo
        mask_p = (lp >= 0) & (lp < HALF)
        mask_s = (ls >= 0) & (ls < HALF) & (sr > 0)
        plsc.store_scatter(a_ref, [jnp.clip(lp, 0, HALF - 1)], key_p, mask=mask_p)
        plsc.store_scatter(a_ref, [jnp.clip(ls, 0, HALF - 1)], key_s, mask=mask_s)
        c0 = jnp.maximum(c0, jnp.max(jnp.where(st_p < lo, key_p, 0)))
        c0 = jnp.maximum(c0, jnp.max(jnp.where((st_s < lo) & (sr > 0), key_s, 0)))
        return cp + jnp.sum(pr), cs + jnp.sum(sr), c0

    cp, cs, c0 = lax.fori_loop(0, L // NLANE, scan_body, (0, 0, 0))
    tgt = cp + cs
    nv = jnp.clip(tgt - lo, 0, HALF)
    nvc = (nv + CH - 1) // CH

    def fill_body(i, cm):
        seg = jnp.maximum(plsc.cummax(a_ref[pl.ds(i * NLANE, NLANE)]), cm)
        tvec = lo + i * NLANE + lane
        row = jnp.where((seg & 1) == 1, 0, seg >> 1)
        gid = b * L + jnp.where(tvec < tgt, row, 0)
        g_ref[i // (CH // NLANE), pl.ds((i % (CH // NLANE)) * NLANE, NLANE)] = gid
        return jnp.max(seg)
    lax.fori_loop(0, HALF // NLANE, fill_body, c0)

    @pl.when(half == 1)
    def _():
        tl_v[...] = jnp.full((NLANE,), tgt, jnp.int32)
        pltpu.sync_copy(tl_v, tl16.at[b])

    obase = b * M + lo

    def gather(c, buf, sem):
        pltpu.async_copy(xf.at[g_ref.at[c]], buf, sem)

    def gwait(c, buf, sem):
        pltpu.make_async_copy(xf.at[g_ref.at[c]], buf, sem).wait()

    @pl.when(nvc > 0)
    def _():
        gather(0, buf0, sem0)

    @pl.when(nvc > 1)
    def _():
        gather(1, buf1, sem1)

    def zero_tail(c, buf):
        def zrow(r, _):
            @pl.when(c * CH + r >= nv)
            def _():
                for j in range(H // NLANE):
                    buf[r, pl.ds(j * NLANE, NLANE)] = zf16
            return 0
        lax.fori_loop(0, CH, zrow, 0)

    def gbody(i, _):
        for k, (buf, sem) in enumerate(((buf0, sem0), (buf1, sem1))):
            c = 2 * i + k

            @pl.when(c < nvc)
            def _():
                gwait(c, buf, sem)

                @pl.when(c == nvc - 1)
                def _():
                    zero_tail(c, buf)
                pltpu.sync_copy(buf, out.at[pl.ds(obase + c * CH, CH)])

                @pl.when(c + 2 < nvc)
                def _():
                    gather(c + 2, buf, sem)

            @pl.when(c >= nvc)
            def _():
                pltpu.sync_copy(zbuf, out.at[pl.ds(obase + c * CH, CH)])
        return 0
    lax.fori_loop(0, NCH // 2, gbody, 0)


@jax.jit
def _run(xf, pd, sd, sl):
    mesh = plsc.VectorSubcoreMesh(core_axis_name="c", subcore_axis_name="s")
    f = pl.kernel(
        _sc_body,
        out_type=(
            jax.ShapeDtypeStruct((B * M, H), jnp.float32),
            jax.ShapeDtypeStruct((16, 16), jnp.int32),
        ),
        mesh=mesh,
        scratch_types=[
            pltpu.VMEM((L,), jnp.int32),
            pltpu.VMEM((L,), jnp.int32),
            pltpu.VMEM((16,), jnp.int32),
            pltpu.VMEM((16,), jnp.int32),
            pltpu.VMEM((HALF,), jnp.int32),
            pltpu.VMEM((NCH, CH), jnp.int32),
            pltpu.VMEM((CH, H), jnp.float32),
            pltpu.VMEM((CH, H), jnp.float32),
            pltpu.VMEM((CH, H), jnp.float32),
            pltpu.SemaphoreType.DMA,
            pltpu.SemaphoreType.DMA,
        ],
        compiler_params=pltpu.CompilerParams(needs_layout_passes=False),
    )
    return f(xf, pd, sd, sl)


def kernel(x, phone_duration, sil_duration, src_lens, max_len):
    out_flat, tl16 = _run(x.reshape(B * L, H), phone_duration,
                          sil_duration, src_lens)
    return out_flat.reshape(B, M, H), tl16[:, 0]


# speedup vs baseline: 67.3138x; 1.0415x over previous
import jax
import jax.numpy as jnp
from jax import lax
from jax.experimental import pallas as pl
from jax.experimental.pallas import tpu as pltpu
from jax.experimental.pallas import tpu_sc as plsc

B, L, H, M = 16, 2048, 256, 8192
HALF = M // 2
CH = 64
NCH = HALF // CH
NLANE = 16
RING = 4


def _sc_body(xf, pd, sd, sl, out, tl16,
             pd_v, sd_v, sl_v, tl_v, a_ref, g_ref, bufs, zbuf,
             gsems, osems, zsem):
    cid = lax.axis_index("c")
    sid = lax.axis_index("s")
    b = sid
    half = (sid + cid) & 1
    lo = half * HALF

    pltpu.sync_copy(pd.at[b], pd_v)
    pltpu.sync_copy(sd.at[b], sd_v)
    pltpu.sync_copy(sl, sl_v)
    lane = lax.iota(jnp.int32, NLANE)
    src_len = jnp.sum(jnp.where(lane == b, sl_v[...], 0))

    zero16 = jnp.zeros((NLANE,), jnp.int32)
    zf16 = jnp.zeros((NLANE,), jnp.float32)

    def init_body(i, _):
        a_ref[pl.ds(i * NLANE, NLANE)] = zero16
        return 0
    lax.fori_loop(0, HALF // NLANE, init_body, 0)

    def zinit_body(i, _):
        zbuf[i // (H // NLANE), pl.ds((i % (H // NLANE)) * NLANE, NLANE)] = zf16
        return 0
    lax.fori_loop(0, CH * H // NLANE, zinit_body, 0)

    def scan_body(i, carry):
        cp, cs, c0 = carry
        lvec = i * NLANE + lane
        valid = lvec < src_len
        pr = jnp.maximum(jnp.where(valid, pd_v[pl.ds(i * NLANE, NLANE)], 0), 1)
        sr = jnp.where(valid, sd_v[pl.ds(i * NLANE, NLANE)], 0)
        P = plsc.cumsum(pr) + cp
        S = plsc.cumsum(sr) + cs
        Pe = P - pr
        Se = S - sr
        key_p = 2 * lvec
        key_s = key_p + 1
        st_p = Pe + Se
        st_s = P + Se
        lp = st_p - lo
        ls = st_s - lo
        mask_p = (lp >= 0) & (lp < HALF)
        mask_s = (ls >= 0) & (ls < HALF) & (sr > 0)
        plsc.store_scatter(a_ref, [jnp.clip(lp, 0, HALF - 1)], key_p, mask=mask_p)
        plsc.store_scatter(a_ref, [jnp.clip(ls, 0, HALF - 1)], key_s, mask=mask_s)
        c0 = jnp.maximum(c0, jnp.max(jnp.where(st_p < lo, key_p, 0)))
        c0 = jnp.maximum(c0, jnp.max(jnp.where((st_s < lo) & (sr > 0), key_s, 0)))
        return cp + jnp.sum(pr), cs + jnp.sum(sr), c0

    cp, cs, c0 = lax.fori_loop(0, L // NLANE, scan_body, (0, 0, 0))
    tgt = cp + cs
    nv = jnp.clip(tgt - lo, 0, HALF)
    nvc = (nv + CH - 1) // CH

    def fill_body(i, cm):
        seg = jnp.maximum(plsc.cummax(a_ref[pl.ds(i * NLANE, NLANE)]), cm)
        tvec = lo + i * NLANE + lane
        row = jnp.where((seg & 1) == 1, 0, seg >> 1)
        gid = b * L + jnp.where(tvec < tgt, row, 0)
        g_ref[i // (CH // NLANE), pl.ds((i % (CH // NLANE)) * NLANE, NLANE)] = gid
        return jnp.max(seg)
    lax.fori_loop(0, HALF // NLANE, fill_body, c0)

    @pl.when(half == 1)
    def _():
        tl_v[...] = jnp.full((NLANE,), tgt, jnp.int32)
        pltpu.sync_copy(tl_v, tl16.at[b])

    obase = b * M + lo

    def gather(c, k):
        pltpu.async_copy(xf.at[g_ref.at[c]], bufs[k], gsems[k])

    def gwait(c, k):
        pltpu.make_async_copy(xf.at[g_ref.at[c]], bufs[k], gsems[k]).wait()

    def owait(k):
        pltpu.make_async_copy(bufs[k], out.at[pl.ds(obase, CH)], osems[k]).wait()

    def zfire(c, _):
        pltpu.async_copy(zbuf, out.at[pl.ds(obase + c * CH, CH)], zsem)
        return 0
    lax.fori_loop(nvc, NCH, zfire, 0)

    @pl.when(nvc > 0)
    def _():
        gather(0, 0)

    @pl.when(nvc > 1)
    def _():
        gather(1, 1)

    def zero_tail(c, k):
        def zrow(r, _):
            @pl.when(c * CH + r >= nv)
            def _():
                for j in range(H // NLANE):
                    bufs[k][r, pl.ds(j * NLANE, NLANE)] = zf16
            return 0
        lax.fori_loop(0, CH, zrow, 0)

    def gbody(i, _):
        for k in range(RING):
            c = RING * i + k

            @pl.when(c < nvc)
            def _():
                gwait(c, k)

                @pl.when(c == nvc - 1)
                def _():
                    zero_tail(c, k)
                pltpu.async_copy(bufs[k], out.at[pl.ds(obase + c * CH, CH)],
                                 osems[k])

                @pl.when(c + 2 < nvc)
                def _():
                    k2 = (k + 2) % RING

                    @pl.when(c >= 2)
                    def _():
                        owait(k2)
                    gather(c + 2, k2)
        return 0
    lax.fori_loop(0, NCH // RING, gbody, 0)

    for k in range(RING):
        @pl.when(nvc > k)
        def _():
            owait(k)

    def zdrain(j, _):
        pltpu.make_async_copy(zbuf, out.at[pl.ds(obase, CH)], zsem).wait()
        return 0
    lax.fori_loop(0, NCH - nvc, zdrain, 0)


@jax.jit
def _run(xf, pd, sd, sl):
    mesh = plsc.VectorSubcoreMesh(core_axis_name="c", subcore_axis_name="s")
    f = pl.kernel(
        _sc_body,
        out_type=(
            jax.ShapeDtypeStruct((B * M, H), jnp.float32),
            jax.ShapeDtypeStruct((16, 16), jnp.int32),
        ),
        mesh=mesh,
        scratch_types=[
            pltpu.VMEM((L,), jnp.int32),
            pltpu.VMEM((L,), jnp.int32),
            pltpu.VMEM((16,), jnp.int32),
            pltpu.VMEM((16,), jnp.int32),
            pltpu.VMEM((HALF,), jnp.int32),
            pltpu.VMEM((NCH, CH), jnp.int32),
            [pltpu.VMEM((CH, H), jnp.float32) for _ in range(RING)],
            pltpu.VMEM((CH, H), jnp.float32),
            [pltpu.SemaphoreType.DMA for _ in range(RING)],
            [pltpu.SemaphoreType.DMA for _ in range(RING)],
            pltpu.SemaphoreType.DMA,
        ],
        compiler_params=pltpu.CompilerParams(needs_layout_passes=False),
    )
    return f(xf, pd, sd, sl)


def kernel(x, phone_duration, sil_duration, src_lens, max_len):
    out_flat, tl16 = _run(x.reshape(B * L, H), phone_duration,
                          sil_duration, src_lens)
    return out_flat.reshape(B, M, H), tl16[:, 0]
